# Initial kernel scaffold; baseline (speedup 1.0000x reference)
import jax
import jax.numpy as jnp
from jax.experimental import pallas as pl


def kernel(x, adj, W1, b1, gamma1, beta1, W2, b2, gamma2, beta2):
    raise NotImplementedError("write your pallas kernel here")


# speedup vs baseline: 1.0099x; 1.0099x over previous
import jax
import jax.numpy as jnp
from jax.experimental import pallas as pl

EPS_ = 1e-5


def _pick_block(n, target):
    for r in (target, 2000, 1000, 400, 200, 100, 40, 8):
        if r <= target and n % r == 0:
            return r
    return n


def _xw_kern(x_ref, w_ref, o_ref):
    o_ref[...] = jnp.dot(x_ref[...], w_ref[...],
                         preferred_element_type=jnp.float32)


def _xw(x, w):
    n, d = x.shape
    dout = w.shape[1]
    r = _pick_block(n, 2000)
    return pl.pallas_call(
        _xw_kern,
        grid=(n // r,),
        in_specs=[pl.BlockSpec((r, d), lambda i: (i, 0)),
                  pl.BlockSpec((d, dout), lambda i: (0, 0))],
        out_specs=pl.BlockSpec((r, dout), lambda i: (i, 0)),
        out_shape=jax.ShapeDtypeStruct((n, dout), jnp.float32),
    )(x, w)


def _affine_xw_kern(h_ref, sc_ref, sh_ref, w_ref, o_ref):
    hh = h_ref[...] * sc_ref[...] + sh_ref[...]
    o_ref[...] = jnp.dot(hh, w_ref[...], preferred_element_type=jnp.float32)


def _affine_xw(h, scale, shift, w):
    n, d = h.shape
    dout = w.shape[1]
    r = _pick_block(n, 2000)
    return pl.pallas_call(
        _affine_xw_kern,
        grid=(n // r,),
        in_specs=[pl.BlockSpec((r, d), lambda i: (i, 0)),
                  pl.BlockSpec((1, d), lambda i: (0, 0)),
                  pl.BlockSpec((1, d), lambda i: (0, 0)),
                  pl.BlockSpec((d, dout), lambda i: (0, 0))],
        out_specs=pl.BlockSpec((r, dout), lambda i: (i, 0)),
        out_shape=jax.ShapeDtypeStruct((n, dout), jnp.float32),
    )(h, scale.reshape(1, d), shift.reshape(1, d), w)


def _adj_mm_kern(adj_ref, s_ref, o_ref, st_ref):
    h = jnp.dot(adj_ref[...], s_ref[...], preferred_element_type=jnp.float32)
    o_ref[...] = h
    st_ref[0, 0:1, :] = jnp.sum(h, axis=0, keepdims=True)
    st_ref[0, 1:2, :] = jnp.sum(h * h, axis=0, keepdims=True)


def _adj_mm(adj, s):
    n = adj.shape[0]
    d = s.shape[1]
    r = _pick_block(n, 400)
    nb = n // r
    h, stats = pl.pallas_call(
        _adj_mm_kern,
        grid=(nb,),
        in_specs=[pl.BlockSpec((r, n), lambda i: (i, 0)),
                  pl.BlockSpec((n, d), lambda i: (0, 0))],
        out_specs=[pl.BlockSpec((r, d), lambda i: (i, 0)),
                   pl.BlockSpec((1, 2, d), lambda i: (i, 0, 0))],
        out_shape=[jax.ShapeDtypeStruct((n, d), jnp.float32),
                   jax.ShapeDtypeStruct((nb, 2, d), jnp.float32)],
    )(adj, s)
    return h, stats


def _tanh_affine_kern(h_ref, sc_ref, sh_ref, o_ref):
    o_ref[...] = jnp.tanh(h_ref[...] * sc_ref[...] + sh_ref[...])


def _tanh_affine(h, scale, shift):
    n, d = h.shape
    r = _pick_block(n, 2000)
    return pl.pallas_call(
        _tanh_affine_kern,
        grid=(n // r,),
        in_specs=[pl.BlockSpec((r, d), lambda i: (i, 0)),
                  pl.BlockSpec((1, d), lambda i: (0, 0)),
                  pl.BlockSpec((1, d), lambda i: (0, 0))],
        out_specs=pl.BlockSpec((r, d), lambda i: (i, 0)),
        out_shape=jax.ShapeDtypeStruct((n, d), jnp.float32),
    )(h, scale.reshape(1, d), shift.reshape(1, d))


def _bn_coeffs(stats, n, b, gamma, beta):
    cs = jnp.sum(stats[:, 0, :], axis=0)
    css = jnp.sum(stats[:, 1, :], axis=0)
    mu = cs / n + b
    ex2 = (css + 2.0 * b * cs) / n + b * b
    var = ex2 - mu * mu
    scale = gamma * jax.lax.rsqrt(var + EPS_)
    shift = (b - mu) * scale + beta
    return scale, shift


def kernel(x, adj, W1, b1, gamma1, beta1, W2, b2, gamma2, beta2):
    n = adj.shape[0]
    support1 = _xw(x, W1)
    h1, stats1 = _adj_mm(adj, support1)
    scale1, shift1 = _bn_coeffs(stats1, n, b1, gamma1, beta1)
    support2 = _affine_xw(h1, scale1, shift1, W2)
    h2, stats2 = _adj_mm(adj, support2)
    scale2, shift2 = _bn_coeffs(stats2, n, b2, gamma2, beta2)
    return _tanh_affine(h2, scale2, shift2)


# speedup vs baseline: 1.0890x; 1.0784x over previous
import jax
import jax.numpy as jnp
from jax.experimental import pallas as pl
from jax.experimental.pallas import tpu as pltpu

EPS_ = 1e-5


def _pick_block(n, target):
    for r in (target, 2000, 1000, 400, 200, 100, 40, 8):
        if r <= target and n % r == 0:
            return r
    return n


def _gcn1_kern(adj_ref, x_ref, w1_ref, o_ref, st_ref, s1_ref):
    i = pl.program_id(0)
    n = x_ref.shape[0]
    c = _pick_block(n, 2000)

    @pl.when(i == 0)
    def _prologue():
        for j in range(n // c):
            sl = pl.ds(j * c, c)
            s1_ref[sl, :] = jnp.dot(x_ref[sl, :], w1_ref[...],
                                    preferred_element_type=jnp.float32)

    h = jnp.dot(adj_ref[...], s1_ref[...], preferred_element_type=jnp.float32)
    o_ref[...] = h.astype(jnp.bfloat16)
    st_ref[0, 0:1, :] = jnp.sum(h, axis=0, keepdims=True)
    st_ref[0, 1:2, :] = jnp.sum(h * h, axis=0, keepdims=True)


def _gcn2_kern(adj_ref, h1_ref, sc_ref, sh_ref, w2_ref, o_ref, st_ref, s2_ref):
    i = pl.program_id(0)
    n = h1_ref.shape[0]
    c = _pick_block(n, 2000)

    @pl.when(i == 0)
    def _prologue():
        for j in range(n // c):
            sl = pl.ds(j * c, c)
            hh = h1_ref[sl, :].astype(jnp.float32) * sc_ref[...] + sh_ref[...]
            s2_ref[sl, :] = jnp.dot(hh, w2_ref[...],
                                    preferred_element_type=jnp.float32)

    h = jnp.dot(adj_ref[...], s2_ref[...], preferred_element_type=jnp.float32)
    o_ref[...] = h.astype(jnp.bfloat16)
    st_ref[0, 0:1, :] = jnp.sum(h, axis=0, keepdims=True)
    st_ref[0, 1:2, :] = jnp.sum(h * h, axis=0, keepdims=True)


def _gcn_layer1(adj, x, w1):
    n = adj.shape[0]
    d = w1.shape[1]
    r = _pick_block(n, 200)
    nb = n // r
    return pl.pallas_call(
        _gcn1_kern,
        grid=(nb,),
        in_specs=[pl.BlockSpec((r, n), lambda i: (i, 0)),
                  pl.BlockSpec((n, x.shape[1]), lambda i: (0, 0)),
                  pl.BlockSpec((x.shape[1], d), lambda i: (0, 0))],
        out_specs=[pl.BlockSpec((r, d), lambda i: (i, 0)),
                   pl.BlockSpec((1, 2, d), lambda i: (i, 0, 0))],
        out_shape=[jax.ShapeDtypeStruct((n, d), jnp.bfloat16),
                   jax.ShapeDtypeStruct((nb, 2, d), jnp.float32)],
        scratch_shapes=[pltpu.VMEM((n, d), jnp.float32)],
    )(adj, x, w1)


def _gcn_layer2(adj, h1, scale1, shift1, w2):
    n = adj.shape[0]
    d = w2.shape[1]
    dh = h1.shape[1]
    r = _pick_block(n, 200)
    nb = n // r
    return pl.pallas_call(
        _gcn2_kern,
        grid=(nb,),
        in_specs=[pl.BlockSpec((r, n), lambda i: (i, 0)),
                  pl.BlockSpec((n, dh), lambda i: (0, 0)),
                  pl.BlockSpec((1, dh), lambda i: (0, 0)),
                  pl.BlockSpec((1, dh), lambda i: (0, 0)),
                  pl.BlockSpec((dh, d), lambda i: (0, 0))],
        out_specs=[pl.BlockSpec((r, d), lambda i: (i, 0)),
                   pl.BlockSpec((1, 2, d), lambda i: (i, 0, 0))],
        out_shape=[jax.ShapeDtypeStruct((n, d), jnp.bfloat16),
                   jax.ShapeDtypeStruct((nb, 2, d), jnp.float32)],
        scratch_shapes=[pltpu.VMEM((n, d), jnp.float32)],
    )(adj, h1, scale1.reshape(1, dh), shift1.reshape(1, dh), w2)


def _tanh_affine_kern(h_ref, sc_ref, sh_ref, o_ref):
    hh = h_ref[...].astype(jnp.float32)
    o_ref[...] = jnp.tanh(hh * sc_ref[...] + sh_ref[...])


def _tanh_affine(h, scale, shift):
    n, d = h.shape
    r = _pick_block(n, 2000)
    return pl.pallas_call(
        _tanh_affine_kern,
        grid=(n // r,),
        in_specs=[pl.BlockSpec((r, d), lambda i: (i, 0)),
                  pl.BlockSpec((1, d), lambda i: (0, 0)),
                  pl.BlockSpec((1, d), lambda i: (0, 0))],
        out_specs=pl.BlockSpec((r, d), lambda i: (i, 0)),
        out_shape=jax.ShapeDtypeStruct((n, d), jnp.float32),
    )(h, scale.reshape(1, d), shift.reshape(1, d))


def _bn_coeffs(stats, n, b, gamma, beta):
    cs = jnp.sum(stats[:, 0, :], axis=0)
    css = jnp.sum(stats[:, 1, :], axis=0)
    mu = cs / n + b
    ex2 = (css + 2.0 * b * cs) / n + b * b
    var = ex2 - mu * mu
    scale = gamma * jax.lax.rsqrt(var + EPS_)
    shift = (b - mu) * scale + beta
    return scale, shift


def kernel(x, adj, W1, b1, gamma1, beta1, W2, b2, gamma2, beta2):
    n = adj.shape[0]
    h1, stats1 = _gcn_layer1(adj, x, W1)
    scale1, shift1 = _bn_coeffs(stats1, n, b1, gamma1, beta1)
    h2, stats2 = _gcn_layer2(adj, h1, scale1, shift1, W2)
    scale2, shift2 = _bn_coeffs(stats2, n, b2, gamma2, beta2)
    return _tanh_affine(h2, scale2, shift2)
